# Initial kernel scaffold; baseline (speedup 1.0000x reference)
import functools

import jax
import jax.numpy as jnp
from jax import lax
from jax.experimental import pallas as pl
from jax.experimental.pallas import tpu as pltpu
from jax.experimental.pallas import tpu_sc as plsc

N_NODES = 10000
NP = 10240
D = 128
K_INT = 1000
N_EDGES = 160000
EP = 163840
B = 1024

NC = 2
NS = 16
NW = NC * NS

_SC_MESH = plsc.VectorSubcoreMesh(core_axis_name="c", subcore_axis_name="s")


def _tables_body(emb_ref, w_ref, m_ref, out_ref):
    e = emb_ref[...]
    dn = (((1,), (1,)), ((), ()))
    out_ref[0:K_INT, :] = lax.dot_general(e, w_ref[...], dn,
                                          preferred_element_type=jnp.float32)
    out_ref[K_INT:2 * K_INT, :] = lax.dot_general(e, m_ref[...], dn,
                                                  preferred_element_type=jnp.float32)


def _make_tables(emb, W, M):
    return pl.pallas_call(
        _tables_body,
        out_shape=jax.ShapeDtypeStruct((2 * K_INT, D), jnp.float32),
    )(emb, W, M)


@functools.partial(
    pl.kernel,
    out_type=jax.ShapeDtypeStruct((NP, D), jnp.float32),
    mesh=_SC_MESH,
    scratch_types=[
        pltpu.VMEM((120,), jnp.int32),
        pltpu.VMEM((120, D), jnp.float32),
        pltpu.VMEM((10, D), jnp.float32),
        pltpu.SemaphoreType.DMA,
    ],
)
def _internal_kernel(tbl_hbm, idx_hbm, out_hbm, idx_v, rows_v, hbuf, sem):
    wid = lax.axis_index("s") * NC + lax.axis_index("c")

    def chunk(j, _):
        base = wid * 320 + j * 10
        pltpu.sync_copy(idx_hbm.at[pl.ds(base * 12, 120)], idx_v)
        pltpu.async_copy(tbl_hbm.at[idx_v], rows_v, sem).wait()

        def node(i, _):
            r0 = 12 * i

            def col(c, _):
                cc = c * 16
                acc = jnp.zeros((16,), jnp.float32)
                for k in range(4):
                    t = (rows_v[r0 + 3 * k, pl.ds(cc, 16)]
                         + rows_v[r0 + 3 * k + 1, pl.ds(cc, 16)]
                         + rows_v[r0 + 3 * k + 2, pl.ds(cc, 16)])
                    acc = acc + jnp.maximum(t, 0.0)
                hbuf[i, pl.ds(cc, 16)] = acc
                return 0

            return lax.fori_loop(0, 8, col, 0)

        lax.fori_loop(0, 10, node, 0)
        pltpu.sync_copy(hbuf, out_hbm.at[pl.ds(base, 10)])
        return 0

    lax.fori_loop(0, 32, chunk, 0)


def _mid_body(h_ref, v_ref, u_ref, msg_ref, hu_ref):
    h = h_ref[...]
    m = jnp.max(h, axis=1, keepdims=True)
    e = jnp.exp(h - m)
    h = e / jnp.sum(e, axis=1, keepdims=True)
    dn = (((1,), (1,)), ((), ()))
    msg_ref[...] = lax.dot_general(h, v_ref[...], dn,
                                   preferred_element_type=jnp.float32)
    hu_ref[...] = lax.dot_general(h, u_ref[...], dn,
                                  preferred_element_type=jnp.float32)


def _mid(h_pre, V, U):
    blk = 1024
    return pl.pallas_call(
        _mid_body,
        grid=(NP // blk,),
        in_specs=[
            pl.BlockSpec((blk, D), lambda i: (i, 0)),
            pl.BlockSpec((D, D), lambda i: (0, 0)),
            pl.BlockSpec((D, D), lambda i: (0, 0)),
        ],
        out_specs=[
            pl.BlockSpec((blk, D), lambda i: (i, 0)),
            pl.BlockSpec((blk, D), lambda i: (i, 0)),
        ],
        out_shape=[
            jax.ShapeDtypeStruct((NP, D), jnp.float32),
            jax.ShapeDtypeStruct((NP, D), jnp.float32),
        ],
    )(h_pre, V, U)


@functools.partial(
    pl.kernel,
    out_type=[
        jax.ShapeDtypeStruct((NP, D), jnp.float32),
        jax.ShapeDtypeStruct((NP, D), jnp.float32),
    ],
    mesh=_SC_MESH,
    scratch_types=[
        pltpu.VMEM((128,), jnp.int32),
        pltpu.VMEM((128,), jnp.int32),
        pltpu.VMEM((128, D), jnp.float32),
        pltpu.VMEM((64, D), jnp.float32),
        pltpu.VMEM_SHARED((NP, D), jnp.float32),
        pltpu.SemaphoreType.DMA,
    ],
)
def _edge_kernel(msg_hbm, src_hbm, dst_hbm, agg0_hbm, agg1_hbm,
                 sidx, didx, rows_v, zbuf, agg_sh, sem):
    cid = lax.axis_index("c")
    sid = lax.axis_index("s")
    wid = sid * NC + cid

    def zrow(r, _):
        def zcol(c, _):
            zbuf[r, pl.ds(c * 16, 16)] = jnp.zeros((16,), jnp.float32)
            return 0
        return lax.fori_loop(0, 8, zcol, 0)

    lax.fori_loop(0, 64, zrow, 0)

    def zcp(j, _):
        pltpu.sync_copy(zbuf, agg_sh.at[pl.ds(sid * 640 + j * 64, 64)])
        return 0

    lax.fori_loop(0, 10, zcp, 0)
    plsc.subcore_barrier()

    def chunk(j, _):
        e0 = wid * 5120 + j * 128
        pltpu.sync_copy(src_hbm.at[pl.ds(e0, 128)], sidx)
        pltpu.sync_copy(dst_hbm.at[pl.ds(e0, 128)], didx)
        pltpu.async_copy(msg_hbm.at[sidx], rows_v, sem).wait()
        pltpu.sync_copy(rows_v, agg_sh.at[didx], add=True)
        return 0

    lax.fori_loop(0, 40, chunk, 0)
    plsc.subcore_barrier()

    @pl.when(cid == 0)
    def _():
        def cp(j, _):
            rs = pl.ds(sid * 640 + j * 128, 128)
            pltpu.sync_copy(agg_sh.at[rs], agg0_hbm.at[rs])
            return 0
        lax.fori_loop(0, 5, cp, 0)

    @pl.when(cid == 1)
    def _():
        def cp(j, _):
            rs = pl.ds(sid * 640 + j * 128, 128)
            pltpu.sync_copy(agg_sh.at[rs], agg1_hbm.at[rs])
            return 0
        lax.fori_loop(0, 5, cp, 0)


@functools.partial(
    pl.kernel,
    out_type=[
        jax.ShapeDtypeStruct((2 * B, D), jnp.float32),
        jax.ShapeDtypeStruct((2 * B, D), jnp.float32),
        jax.ShapeDtypeStruct((2 * B, D), jnp.float32),
    ],
    mesh=_SC_MESH,
    scratch_types=[
        pltpu.VMEM((64,), jnp.int32),
        pltpu.VMEM((64, D), jnp.float32),
        pltpu.VMEM((64, D), jnp.float32),
        pltpu.VMEM((64, D), jnp.float32),
        pltpu.SemaphoreType.DMA,
    ],
)
def _batch_gather_kernel(hu_hbm, agg0_hbm, agg1_hbm, bidx_hbm,
                         g0_hbm, g1_hbm, g2_hbm, idx_v, b0, b1, b2, sem):
    wid = lax.axis_index("s") * NC + lax.axis_index("c")
    base = wid * 64
    pltpu.sync_copy(bidx_hbm.at[pl.ds(base, 64)], idx_v)
    pltpu.async_copy(hu_hbm.at[idx_v], b0, sem).wait()
    pltpu.async_copy(agg0_hbm.at[idx_v], b1, sem).wait()
    pltpu.async_copy(agg1_hbm.at[idx_v], b2, sem).wait()
    pltpu.sync_copy(b0, g0_hbm.at[pl.ds(base, 64)])
    pltpu.sync_copy(b1, g1_hbm.at[pl.ds(base, 64)])
    pltpu.sync_copy(b2, g2_hbm.at[pl.ds(base, 64)])


def _head_body(g0_ref, g1_ref, g2_ref, l1w_ref, l1b_ref, l2w_ref, l2b_ref,
               out_ref):
    r = jnp.maximum(g0_ref[...] + g1_ref[...] + g2_ref[...], 0.0)
    m = jnp.max(r, axis=1, keepdims=True)
    e = jnp.exp(r - m)
    r = e / jnp.sum(e, axis=1, keepdims=True)
    a = r[0:B]
    b = r[B:2 * B]
    w1 = l1w_ref[...]
    dn = (((1,), (1,)), ((), ()))
    x = (lax.dot_general(a, w1[:, 0:D], dn, preferred_element_type=jnp.float32)
         + lax.dot_general(b, w1[:, D:2 * D], dn,
                           preferred_element_type=jnp.float32)
         + l1b_ref[...][None, :])
    x = jnp.where(x > 0, x, 0.01 * x)
    logits = lax.dot_general(x, l2w_ref[...], dn,
                             preferred_element_type=jnp.float32) \
        + l2b_ref[...][None, :]
    m2 = jnp.max(logits, axis=1, keepdims=True)
    e2 = jnp.exp(logits - m2)
    out_ref[...] = e2 / jnp.sum(e2, axis=1, keepdims=True)


def _head(g0, g1, g2, L1_w, L1_b, L2_w, L2_b):
    return pl.pallas_call(
        _head_body,
        out_shape=jax.ShapeDtypeStruct((B, 2), jnp.float32),
    )(g0, g1, g2, L1_w, L1_b, L2_w, L2_b)


@jax.jit
def kernel(batch, internal_node_ids, internal_adj, edge_index, emb, W, M, U, V,
           L1_w, L1_b, L2_w, L2_b):
    ids = internal_node_ids.astype(jnp.int32)
    adj = internal_adj.astype(jnp.int32)
    idx_comb = jnp.concatenate([ids[:, :, None], adj + K_INT], axis=2)
    idx_comb = idx_comb.reshape(N_NODES, 12)
    idx_comb = jnp.pad(idx_comb, ((0, NP - N_NODES), (0, 0))).reshape(-1)

    src = edge_index[0].astype(jnp.int32)
    dst = edge_index[1].astype(jnp.int32)
    srcp = jnp.pad(src, (0, EP - N_EDGES))
    dstp = jnp.pad(dst, (0, EP - N_EDGES), constant_values=NP - 1)

    bidx = jnp.concatenate([batch[:, 0], batch[:, 1]]).astype(jnp.int32)

    tbl = _make_tables(emb, W, M)
    h_pre = _internal_kernel(tbl, idx_comb)
    msg, hu = _mid(h_pre, V, U)
    agg0, agg1 = _edge_kernel(msg, srcp, dstp)
    g0, g1, g2 = _batch_gather_kernel(hu, agg0, agg1, bidx)
    return _head(g0, g1, g2, L1_w, L1_b, L2_w, L2_b)


# speedup vs baseline: 2.3481x; 2.3481x over previous
import functools

import jax
import jax.numpy as jnp
from jax import lax
from jax.experimental import pallas as pl
from jax.experimental.pallas import tpu as pltpu
from jax.experimental.pallas import tpu_sc as plsc

N_NODES = 10000
NP = 10240
D = 128
K_INT = 1000
N_EDGES = 160000
EP = 163840
B = 1024

NC = 2
NS = 16
NW = NC * NS

_SC_MESH = plsc.VectorSubcoreMesh(core_axis_name="c", subcore_axis_name="s")


def _tables_body(emb_ref, w_ref, m_ref, out_ref):
    e = emb_ref[...]
    dn = (((1,), (1,)), ((), ()))
    out_ref[0:K_INT, :] = lax.dot_general(e, w_ref[...], dn,
                                          preferred_element_type=jnp.float32)
    out_ref[K_INT:2 * K_INT, :] = lax.dot_general(e, m_ref[...], dn,
                                                  preferred_element_type=jnp.float32)


def _make_tables(emb, W, M):
    return pl.pallas_call(
        _tables_body,
        out_shape=jax.ShapeDtypeStruct((2 * K_INT, D), jnp.float32),
    )(emb, W, M)


@functools.partial(
    pl.kernel,
    out_type=jax.ShapeDtypeStruct((NP, D), jnp.float32),
    mesh=_SC_MESH,
    scratch_types=[
        pltpu.VMEM((96,), jnp.int32),
        pltpu.VMEM((96, D), jnp.float32),
        pltpu.VMEM((8, D), jnp.float32),
        pltpu.SemaphoreType.DMA,
    ],
)
def _internal_kernel(tbl_hbm, idx_hbm, out_hbm, idx_v, rows_v, hbuf, sem):
    wid = lax.axis_index("s") * NC + lax.axis_index("c")

    def chunk(j, _):
        base = wid * 320 + j * 8
        pltpu.sync_copy(idx_hbm.at[pl.ds(base * 12, 96)], idx_v)
        pltpu.async_copy(tbl_hbm.at[idx_v], rows_v, sem).wait()

        def node(i, _):
            r0 = 12 * i

            def col(c, _):
                cc = c * 16
                acc = jnp.zeros((16,), jnp.float32)
                for k in range(4):
                    t = (rows_v[r0 + 3 * k, pl.ds(cc, 16)]
                         + rows_v[r0 + 3 * k + 1, pl.ds(cc, 16)]
                         + rows_v[r0 + 3 * k + 2, pl.ds(cc, 16)])
                    acc = acc + jnp.maximum(t, 0.0)
                hbuf[i, pl.ds(cc, 16)] = acc
                return 0

            return lax.fori_loop(0, 8, col, 0)

        lax.fori_loop(0, 8, node, 0)
        pltpu.sync_copy(hbuf, out_hbm.at[pl.ds(base, 8)])
        return 0

    lax.fori_loop(0, 40, chunk, 0)


def _mid_body(h_ref, v_ref, u_ref, msg_ref, hu_ref):
    h = h_ref[...]
    m = jnp.max(h, axis=1, keepdims=True)
    e = jnp.exp(h - m)
    h = e / jnp.sum(e, axis=1, keepdims=True)
    dn = (((1,), (1,)), ((), ()))
    msg_ref[...] = lax.dot_general(h, v_ref[...], dn,
                                   preferred_element_type=jnp.float32)
    hu_ref[...] = lax.dot_general(h, u_ref[...], dn,
                                  preferred_element_type=jnp.float32)


def _mid(h_pre, V, U):
    blk = 1024
    return pl.pallas_call(
        _mid_body,
        grid=(NP // blk,),
        in_specs=[
            pl.BlockSpec((blk, D), lambda i: (i, 0)),
            pl.BlockSpec((D, D), lambda i: (0, 0)),
            pl.BlockSpec((D, D), lambda i: (0, 0)),
        ],
        out_specs=[
            pl.BlockSpec((blk, D), lambda i: (i, 0)),
            pl.BlockSpec((blk, D), lambda i: (i, 0)),
        ],
        out_shape=[
            jax.ShapeDtypeStruct((NP, D), jnp.float32),
            jax.ShapeDtypeStruct((NP, D), jnp.float32),
        ],
    )(h_pre, V, U)


@functools.partial(
    pl.kernel,
    out_type=[
        jax.ShapeDtypeStruct((NP, D), jnp.float32),
        jax.ShapeDtypeStruct((NP, D), jnp.float32),
    ],
    mesh=_SC_MESH,
    scratch_types=[
        pltpu.VMEM((128,), jnp.int32),
        pltpu.VMEM((128,), jnp.int32),
        pltpu.VMEM((128, D), jnp.float32),
        pltpu.VMEM((64, D), jnp.float32),
        pltpu.VMEM_SHARED((NP, D), jnp.float32),
        pltpu.SemaphoreType.DMA,
    ],
)
def _edge_kernel(msg_hbm, src_hbm, dst_hbm, agg0_hbm, agg1_hbm,
                 sidx, didx, rows_v, zbuf, agg_sh, sem):
    cid = lax.axis_index("c")
    sid = lax.axis_index("s")
    wid = sid * NC + cid

    def zrow(r, _):
        def zcol(c, _):
            zbuf[r, pl.ds(c * 16, 16)] = jnp.zeros((16,), jnp.float32)
            return 0
        return lax.fori_loop(0, 8, zcol, 0)

    lax.fori_loop(0, 64, zrow, 0)

    def zcp(j, _):
        pltpu.sync_copy(zbuf, agg_sh.at[pl.ds(sid * 640 + j * 64, 64)])
        return 0

    lax.fori_loop(0, 10, zcp, 0)
    plsc.subcore_barrier()

    def chunk(j, _):
        e0 = wid * 5120 + j * 128
        pltpu.sync_copy(src_hbm.at[pl.ds(e0, 128)], sidx)
        pltpu.sync_copy(dst_hbm.at[pl.ds(e0, 128)], didx)
        pltpu.async_copy(msg_hbm.at[sidx], rows_v, sem).wait()
        pltpu.sync_copy(rows_v, agg_sh.at[didx], add=True)
        return 0

    lax.fori_loop(0, 40, chunk, 0)
    plsc.subcore_barrier()

    @pl.when(cid == 0)
    def _():
        def cp(j, _):
            rs = pl.ds(sid * 640 + j * 128, 128)
            pltpu.sync_copy(agg_sh.at[rs], agg0_hbm.at[rs])
            return 0
        lax.fori_loop(0, 5, cp, 0)

    @pl.when(cid == 1)
    def _():
        def cp(j, _):
            rs = pl.ds(sid * 640 + j * 128, 128)
            pltpu.sync_copy(agg_sh.at[rs], agg1_hbm.at[rs])
            return 0
        lax.fori_loop(0, 5, cp, 0)


@functools.partial(
    pl.kernel,
    out_type=[
        jax.ShapeDtypeStruct((2 * B, D), jnp.float32),
        jax.ShapeDtypeStruct((2 * B, D), jnp.float32),
        jax.ShapeDtypeStruct((2 * B, D), jnp.float32),
    ],
    mesh=_SC_MESH,
    scratch_types=[
        pltpu.VMEM((64,), jnp.int32),
        pltpu.VMEM((64, D), jnp.float32),
        pltpu.VMEM((64, D), jnp.float32),
        pltpu.VMEM((64, D), jnp.float32),
        pltpu.SemaphoreType.DMA,
    ],
)
def _batch_gather_kernel(hu_hbm, agg0_hbm, agg1_hbm, bidx_hbm,
                         g0_hbm, g1_hbm, g2_hbm, idx_v, b0, b1, b2, sem):
    wid = lax.axis_index("s") * NC + lax.axis_index("c")
    base = wid * 64
    pltpu.sync_copy(bidx_hbm.at[pl.ds(base, 64)], idx_v)
    pltpu.async_copy(hu_hbm.at[idx_v], b0, sem).wait()
    pltpu.async_copy(agg0_hbm.at[idx_v], b1, sem).wait()
    pltpu.async_copy(agg1_hbm.at[idx_v], b2, sem).wait()
    pltpu.sync_copy(b0, g0_hbm.at[pl.ds(base, 64)])
    pltpu.sync_copy(b1, g1_hbm.at[pl.ds(base, 64)])
    pltpu.sync_copy(b2, g2_hbm.at[pl.ds(base, 64)])


def _head_body(g0_ref, g1_ref, g2_ref, l1w_ref, l1b_ref, l2w_ref, l2b_ref,
               out_ref):
    r = jnp.maximum(g0_ref[...] + g1_ref[...] + g2_ref[...], 0.0)
    m = jnp.max(r, axis=1, keepdims=True)
    e = jnp.exp(r - m)
    r = e / jnp.sum(e, axis=1, keepdims=True)
    a = r[0:B]
    b = r[B:2 * B]
    w1 = l1w_ref[...]
    dn = (((1,), (1,)), ((), ()))
    x = (lax.dot_general(a, w1[:, 0:D], dn, preferred_element_type=jnp.float32)
         + lax.dot_general(b, w1[:, D:2 * D], dn,
                           preferred_element_type=jnp.float32)
         + l1b_ref[...][None, :])
    x = jnp.where(x > 0, x, 0.01 * x)
    logits = lax.dot_general(x, l2w_ref[...], dn,
                             preferred_element_type=jnp.float32) \
        + l2b_ref[...][None, :]
    m2 = jnp.max(logits, axis=1, keepdims=True)
    e2 = jnp.exp(logits - m2)
    out_ref[...] = e2 / jnp.sum(e2, axis=1, keepdims=True)


def _head(g0, g1, g2, L1_w, L1_b, L2_w, L2_b):
    return pl.pallas_call(
        _head_body,
        out_shape=jax.ShapeDtypeStruct((B, 2), jnp.float32),
    )(g0, g1, g2, L1_w, L1_b, L2_w, L2_b)


@jax.jit
def kernel(batch, internal_node_ids, internal_adj, edge_index, emb, W, M, U, V,
           L1_w, L1_b, L2_w, L2_b):
    ids = internal_node_ids.astype(jnp.int32)
    adj = internal_adj.astype(jnp.int32)
    idx_comb = jnp.concatenate([ids[:, :, None], adj + K_INT], axis=2)
    idx_comb = idx_comb.reshape(N_NODES, 12)
    idx_comb = jnp.pad(idx_comb, ((0, NP - N_NODES), (0, 0))).reshape(-1)

    src = edge_index[0].astype(jnp.int32)
    dst = edge_index[1].astype(jnp.int32)
    srcp = jnp.pad(src, (0, EP - N_EDGES))
    dstp = jnp.pad(dst, (0, EP - N_EDGES), constant_values=NP - 1)

    bidx = jnp.concatenate([batch[:, 0], batch[:, 1]]).astype(jnp.int32)

    tbl = _make_tables(emb, W, M)
    h_pre = _internal_kernel(tbl, idx_comb)
    msg, hu = _mid(h_pre, V, U)
    agg0, agg1 = _edge_kernel(msg, srcp, dstp)
    g0, g1, g2 = _batch_gather_kernel(hu, agg0, agg1, bidx)
    return _head(g0, g1, g2, L1_w, L1_b, L2_w, L2_b)


# speedup vs baseline: 2.8242x; 1.2028x over previous
import functools

import jax
import jax.numpy as jnp
from jax import lax
from jax.experimental import pallas as pl
from jax.experimental.pallas import tpu as pltpu
from jax.experimental.pallas import tpu_sc as plsc

N_NODES = 10000
NP = 10240
D = 128
K_INT = 1000
N_EDGES = 160000
EP = 163840
B = 1024

NC = 2
NS = 16
NW = NC * NS

_SC_MESH = plsc.VectorSubcoreMesh(core_axis_name="c", subcore_axis_name="s")


def _tables_body(emb_ref, w_ref, m_ref, out_ref):
    e = emb_ref[...]
    dn = (((1,), (1,)), ((), ()))
    out_ref[0:K_INT, :] = lax.dot_general(e, w_ref[...], dn,
                                          preferred_element_type=jnp.float32)
    out_ref[K_INT:2 * K_INT, :] = lax.dot_general(e, m_ref[...], dn,
                                                  preferred_element_type=jnp.float32)


def _make_tables(emb, W, M):
    return pl.pallas_call(
        _tables_body,
        out_shape=jax.ShapeDtypeStruct((2 * K_INT, D), jnp.float32),
    )(emb, W, M)


@functools.partial(
    pl.kernel,
    out_type=jax.ShapeDtypeStruct((NP, D), jnp.float32),
    mesh=_SC_MESH,
    scratch_types=[
        pltpu.VMEM((40, 96), jnp.int32),
        pltpu.VMEM((4, 96, D), jnp.float32),
        pltpu.VMEM((32, D), jnp.float32),
        pltpu.SemaphoreType.DMA,
        pltpu.SemaphoreType.DMA,
        pltpu.SemaphoreType.DMA,
        pltpu.SemaphoreType.DMA,
    ],
)
def _internal_kernel(tbl_hbm, idx_hbm, out_hbm, idx_v, rows_v, hbuf,
                     s0, s1, s2, s3):
    wid = lax.axis_index("s") * NC + lax.axis_index("c")
    sems = [s0, s1, s2, s3]
    pltpu.sync_copy(idx_hbm.at[pl.ds(wid * 40, 40)], idx_v)
    for b in range(4):
        pltpu.async_copy(tbl_hbm.at[idx_v.at[b]], rows_v.at[b], sems[b])

    def group(g, _):
        for b in range(4):
            j = g * 4 + b
            pltpu.make_async_copy(tbl_hbm.at[idx_v.at[b]], rows_v.at[b],
                                  sems[b]).wait()

            def node(i, _):
                r0 = 12 * i
                for c in range(8):
                    cc = c * 16
                    acc = jnp.zeros((16,), jnp.float32)
                    for k in range(4):
                        t = (rows_v[b, r0 + 3 * k, pl.ds(cc, 16)]
                             + rows_v[b, r0 + 3 * k + 1, pl.ds(cc, 16)]
                             + rows_v[b, r0 + 3 * k + 2, pl.ds(cc, 16)])
                        acc = acc + jnp.maximum(t, 0.0)
                    hbuf[b * 8 + i, pl.ds(cc, 16)] = acc
                return 0

            lax.fori_loop(0, 8, node, 0)

            @pl.when(j + 4 < 40)
            def _():
                pltpu.async_copy(tbl_hbm.at[idx_v.at[j + 4]],
                                 rows_v.at[b], sems[b])

        pltpu.sync_copy(hbuf, out_hbm.at[pl.ds(wid * 320 + g * 32, 32)])
        return 0

    lax.fori_loop(0, 10, group, 0)


def _mid_body(h_ref, v_ref, u_ref, msg_ref, hu_ref):
    h = h_ref[...]
    m = jnp.max(h, axis=1, keepdims=True)
    e = jnp.exp(h - m)
    h = e / jnp.sum(e, axis=1, keepdims=True)
    dn = (((1,), (1,)), ((), ()))
    msg_ref[...] = lax.dot_general(h, v_ref[...], dn,
                                   preferred_element_type=jnp.float32)
    hu_ref[...] = lax.dot_general(h, u_ref[...], dn,
                                  preferred_element_type=jnp.float32)


def _mid(h_pre, V, U):
    blk = 1024
    return pl.pallas_call(
        _mid_body,
        grid=(NP // blk,),
        in_specs=[
            pl.BlockSpec((blk, D), lambda i: (i, 0)),
            pl.BlockSpec((D, D), lambda i: (0, 0)),
            pl.BlockSpec((D, D), lambda i: (0, 0)),
        ],
        out_specs=[
            pl.BlockSpec((blk, D), lambda i: (i, 0)),
            pl.BlockSpec((blk, D), lambda i: (i, 0)),
        ],
        out_shape=[
            jax.ShapeDtypeStruct((NP, D), jnp.float32),
            jax.ShapeDtypeStruct((NP, D), jnp.float32),
        ],
    )(h_pre, V, U)


@functools.partial(
    pl.kernel,
    out_type=[
        jax.ShapeDtypeStruct((2 * B, D), jnp.float32),
        jax.ShapeDtypeStruct((2 * B, D), jnp.float32),
        jax.ShapeDtypeStruct((2 * B, D), jnp.float32),
    ],
    mesh=_SC_MESH,
    scratch_types=[
        pltpu.VMEM((40, 128), jnp.int32),
        pltpu.VMEM((40, 128), jnp.int32),
        pltpu.VMEM((2, 128, D), jnp.float32),
        pltpu.VMEM((32, D), jnp.float32),
        pltpu.VMEM((128,), jnp.int32),
        pltpu.VMEM_SHARED((NP, D), jnp.float32),
        pltpu.SemaphoreType.DMA,
        pltpu.SemaphoreType.DMA,
    ],
)
def _edge_kernel(msg_hbm, src_hbm, dst_hbm, bidx_hbm, hu_hbm,
                 ghu_hbm, gagg0_hbm, gagg1_hbm,
                 sidx, didx, rows_v, zbuf, bidx_v, agg_sh, s0, s1):
    cid = lax.axis_index("c")
    sid = lax.axis_index("s")
    wid = sid * NC + cid
    sems = [s0, s1]

    pltpu.sync_copy(src_hbm.at[pl.ds(wid * 40, 40)], sidx)
    pltpu.sync_copy(dst_hbm.at[pl.ds(wid * 40, 40)], didx)

    def zrow(r, _):
        for c in range(8):
            zbuf[r, pl.ds(c * 16, 16)] = jnp.zeros((16,), jnp.float32)
        return 0

    lax.fori_loop(0, 32, zrow, 0)

    def zcp(j, _):
        pltpu.sync_copy(zbuf, agg_sh.at[pl.ds(sid * 640 + j * 32, 32)])
        return 0

    lax.fori_loop(0, 20, zcp, 0)
    plsc.subcore_barrier()

    for b in range(2):
        pltpu.async_copy(msg_hbm.at[sidx.at[b]], rows_v.at[b], sems[b])

    def group(g, _):
        for b in range(2):
            j = g * 2 + b
            pltpu.make_async_copy(msg_hbm.at[sidx.at[b]], rows_v.at[b],
                                  sems[b]).wait()
            pltpu.sync_copy(rows_v.at[b], agg_sh.at[didx.at[j]], add=True)

            @pl.when(j + 2 < 40)
            def _():
                pltpu.async_copy(msg_hbm.at[sidx.at[j + 2]],
                                 rows_v.at[b], sems[b])
        return 0

    lax.fori_loop(0, 20, group, 0)

    @pl.when(cid == 0)
    def _():
        pltpu.sync_copy(bidx_hbm.at[pl.ds(sid * 128, 128)], bidx_v)
        pltpu.async_copy(hu_hbm.at[bidx_v], rows_v.at[0], s0).wait()
        pltpu.sync_copy(rows_v.at[0], ghu_hbm.at[pl.ds(sid * 128, 128)])

    plsc.subcore_barrier()

    pltpu.sync_copy(bidx_hbm.at[pl.ds(sid * 128, 128)], bidx_v)
    pltpu.async_copy(agg_sh.at[bidx_v], rows_v.at[1], s1).wait()

    @pl.when(cid == 0)
    def _():
        pltpu.sync_copy(rows_v.at[1], gagg0_hbm.at[pl.ds(sid * 128, 128)])

    @pl.when(cid == 1)
    def _():
        pltpu.sync_copy(rows_v.at[1], gagg1_hbm.at[pl.ds(sid * 128, 128)])


def _head_body(g0_ref, g1_ref, g2_ref, l1w_ref, l1b_ref, l2w_ref, l2b_ref,
               out_ref):
    r = jnp.maximum(g0_ref[...] + g1_ref[...] + g2_ref[...], 0.0)
    m = jnp.max(r, axis=1, keepdims=True)
    e = jnp.exp(r - m)
    r = e / jnp.sum(e, axis=1, keepdims=True)
    a = r[0:B]
    b = r[B:2 * B]
    w1 = l1w_ref[...]
    dn = (((1,), (1,)), ((), ()))
    x = (lax.dot_general(a, w1[:, 0:D], dn, preferred_element_type=jnp.float32)
         + lax.dot_general(b, w1[:, D:2 * D], dn,
                           preferred_element_type=jnp.float32)
         + l1b_ref[...][None, :])
    x = jnp.where(x > 0, x, 0.01 * x)
    logits = lax.dot_general(x, l2w_ref[...], dn,
                             preferred_element_type=jnp.float32) \
        + l2b_ref[...][None, :]
    m2 = jnp.max(logits, axis=1, keepdims=True)
    e2 = jnp.exp(logits - m2)
    out_ref[...] = e2 / jnp.sum(e2, axis=1, keepdims=True)


def _head(g0, g1, g2, L1_w, L1_b, L2_w, L2_b):
    return pl.pallas_call(
        _head_body,
        out_shape=jax.ShapeDtypeStruct((B, 2), jnp.float32),
    )(g0, g1, g2, L1_w, L1_b, L2_w, L2_b)


@jax.jit
def kernel(batch, internal_node_ids, internal_adj, edge_index, emb, W, M, U, V,
           L1_w, L1_b, L2_w, L2_b):
    ids = internal_node_ids.astype(jnp.int32)
    adj = internal_adj.astype(jnp.int32)
    idx_comb = jnp.concatenate([ids[:, :, None], adj + K_INT], axis=2)
    idx_comb = idx_comb.reshape(N_NODES, 12)
    idx_comb = jnp.pad(idx_comb, ((0, NP - N_NODES), (0, 0)))
    idx_comb = idx_comb.reshape(NP * 12 // 96, 96)

    src = edge_index[0].astype(jnp.int32)
    dst = edge_index[1].astype(jnp.int32)
    srcp = jnp.pad(src, (0, EP - N_EDGES)).reshape(EP // 128, 128)
    dstp = jnp.pad(dst, (0, EP - N_EDGES),
                   constant_values=NP - 1).reshape(EP // 128, 128)

    bidx = jnp.concatenate([batch[:, 0], batch[:, 1]]).astype(jnp.int32)

    tbl = _make_tables(emb, W, M)
    h_pre = _internal_kernel(tbl, idx_comb)
    msg, hu = _mid(h_pre, V, U)
    g0, g1, g2 = _edge_kernel(msg, srcp, dstp, bidx, hu)
    return _head(g0, g1, g2, L1_w, L1_b, L2_w, L2_b)


# speedup vs baseline: 9.8462x; 3.4863x over previous
import functools

import jax
import jax.numpy as jnp
from jax import lax
from jax.experimental import pallas as pl
from jax.experimental.pallas import tpu as pltpu
from jax.experimental.pallas import tpu_sc as plsc

N_NODES = 10000
NP = 10240
D = 128
K_INT = 1000
N_EDGES = 160000
EP = 163840
B = 1024
AGG_ROWS = 10112

NC = 2
NS = 16
NW = NC * NS

_SC_MESH = plsc.VectorSubcoreMesh(core_axis_name="c", subcore_axis_name="s")


def _tables_body(emb_ref, w_ref, m_ref, out_ref):
    e = emb_ref[...]
    dn = (((1,), (1,)), ((), ()))
    out_ref[0:K_INT, :] = lax.dot_general(e, w_ref[...], dn,
                                          preferred_element_type=jnp.float32)
    out_ref[K_INT:2 * K_INT, :] = lax.dot_general(e, m_ref[...], dn,
                                                  preferred_element_type=jnp.float32)


def _make_tables(emb, W, M):
    return pl.pallas_call(
        _tables_body,
        out_shape=jax.ShapeDtypeStruct((2 * K_INT, D), jnp.float32),
    )(emb, W, M)


@functools.partial(
    pl.kernel,
    out_type=jax.ShapeDtypeStruct((NP, D), jnp.float32),
    mesh=_SC_MESH,
    scratch_types=[
        pltpu.VMEM((40, 96), jnp.int32),
        pltpu.VMEM((4, 96, D), jnp.float32),
        pltpu.VMEM((32, D), jnp.float32),
        pltpu.VMEM_SHARED((2 * K_INT, D), jnp.float32),
        pltpu.SemaphoreType.DMA,
        pltpu.SemaphoreType.DMA,
        pltpu.SemaphoreType.DMA,
        pltpu.SemaphoreType.DMA,
    ],
)
def _internal_kernel(tbl_hbm, idx_hbm, out_hbm, idx_v, rows_v, hbuf, tbl_sh,
                     s0, s1, s2, s3):
    sid = lax.axis_index("s")
    wid = sid * NC + lax.axis_index("c")
    sems = [s0, s1, s2, s3]
    pltpu.sync_copy(idx_hbm.at[pl.ds(wid * 40, 40)], idx_v)
    tb = jnp.minimum(sid * 128, 2 * K_INT - 128)
    pltpu.sync_copy(tbl_hbm.at[pl.ds(tb, 128)], tbl_sh.at[pl.ds(tb, 128)])
    plsc.subcore_barrier()
    for b in range(4):
        pltpu.async_copy(tbl_sh.at[idx_v.at[b]], rows_v.at[b], sems[b])

    def group(g, _):
        for b in range(4):
            j = g * 4 + b
            pltpu.make_async_copy(tbl_sh.at[idx_v.at[b]], rows_v.at[b],
                                  sems[b]).wait()

            def col(c, _):
                cc = c * 16
                for i in range(8):
                    acc = jnp.zeros((16,), jnp.float32)
                    for k in range(4):
                        t = (rows_v[b, 12 * i + 3 * k, pl.ds(cc, 16)]
                             + rows_v[b, 12 * i + 3 * k + 1, pl.ds(cc, 16)]
                             + rows_v[b, 12 * i + 3 * k + 2, pl.ds(cc, 16)])
                        acc = acc + jnp.maximum(t, 0.0)
                    hbuf[b * 8 + i, pl.ds(cc, 16)] = acc
                return 0

            lax.fori_loop(0, 8, col, 0)

            @pl.when(j + 4 < 40)
            def _():
                pltpu.async_copy(tbl_sh.at[idx_v.at[j + 4]],
                                 rows_v.at[b], sems[b])

        pltpu.sync_copy(hbuf, out_hbm.at[pl.ds(wid * 320 + g * 32, 32)])
        return 0

    lax.fori_loop(0, 10, group, 0)


def _mid_body(h_ref, v_ref, u_ref, msg_ref, hu_ref):
    h = h_ref[...]
    m = jnp.max(h, axis=1, keepdims=True)
    e = jnp.exp(h - m)
    h = e / jnp.sum(e, axis=1, keepdims=True)
    dn = (((1,), (1,)), ((), ()))
    msg = lax.dot_general(h, v_ref[...], dn, preferred_element_type=jnp.float32)
    rows = pl.program_id(0) * h.shape[0] \
        + lax.broadcasted_iota(jnp.int32, h.shape, 0)
    msg_ref[...] = jnp.where(rows < N_NODES, msg, 0.0)
    hu_ref[...] = lax.dot_general(h, u_ref[...], dn,
                                  preferred_element_type=jnp.float32)


def _mid(h_pre, V, U):
    blk = 1024
    return pl.pallas_call(
        _mid_body,
        grid=(NP // blk,),
        in_specs=[
            pl.BlockSpec((blk, D), lambda i: (i, 0)),
            pl.BlockSpec((D, D), lambda i: (0, 0)),
            pl.BlockSpec((D, D), lambda i: (0, 0)),
        ],
        out_specs=[
            pl.BlockSpec((blk, D), lambda i: (i, 0)),
            pl.BlockSpec((blk, D), lambda i: (i, 0)),
        ],
        out_shape=[
            jax.ShapeDtypeStruct((NP, D), jnp.float32),
            jax.ShapeDtypeStruct((NP, D), jnp.float32),
        ],
    )(h_pre, V, U)


@functools.partial(
    pl.kernel,
    out_type=[
        jax.ShapeDtypeStruct((2 * B, D), jnp.float32),
        jax.ShapeDtypeStruct((2 * B, D), jnp.float32),
    ],
    mesh=_SC_MESH,
    scratch_types=[
        pltpu.VMEM((40, 128), jnp.int32),
        pltpu.VMEM((40, 128), jnp.int32),
        pltpu.VMEM((2, 128, D), jnp.float32),
        pltpu.VMEM((1, 128), jnp.int32),
        pltpu.VMEM_SHARED((AGG_ROWS, D), jnp.float32),
        pltpu.SemaphoreType.DMA,
        pltpu.SemaphoreType.DMA,
        pltpu.SemaphoreType.DMA,
    ],
)
def _edge_kernel(msg_hbm, src_hbm, dst_hbm, bidx_hbm, hu_hbm,
                 gagg0_hbm, gagg1_hbm,
                 sidx, didx, rows_v, bidx_v, agg_sh, s0, s1, s2):
    cid = lax.axis_index("c")
    sid = lax.axis_index("s")
    wid = sid * NC + cid
    sems = [s0, s1]

    pltpu.sync_copy(src_hbm.at[pl.ds(wid * 40, 40)], sidx)
    pltpu.sync_copy(dst_hbm.at[pl.ds(wid * 40, 40)], didx)

    def zrow(r, _):
        for c in range(8):
            rows_v[0, r, pl.ds(c * 16, 16)] = jnp.zeros((16,), jnp.float32)
        return 0

    with jax.named_scope("edge_zero"):
        @pl.when(cid == 0)
        def _():
            for k in range(4):
                rs = pl.ds(sid * 632 + k * 128, 128)
                pltpu.sync_copy(hu_hbm.at[rs], agg_sh.at[rs])
            rs = pl.ds(sid * 632 + 504, 128)
            pltpu.sync_copy(hu_hbm.at[rs], agg_sh.at[rs])

        @pl.when(cid == 1)
        def _():
            lax.fori_loop(0, 128, zrow, 0)
            for k in range(4):
                pltpu.sync_copy(rows_v.at[0],
                                agg_sh.at[pl.ds(sid * 632 + k * 128, 128)])
            pltpu.sync_copy(rows_v.at[0],
                            agg_sh.at[pl.ds(sid * 632 + 504, 128)])

        plsc.subcore_barrier()

    with jax.named_scope("edge_loop"):
        for b in range(2):
            pltpu.async_copy(msg_hbm.at[sidx.at[b]], rows_v.at[b], sems[b])

        def group(g, _):
            for b in range(2):
                j = g * 2 + b
                pltpu.make_async_copy(msg_hbm.at[sidx.at[b]], rows_v.at[b],
                                      sems[b]).wait()
                pltpu.sync_copy(rows_v.at[b], agg_sh.at[didx.at[j]],
                                add=True)

                @pl.when(j + 2 < 40)
                def _():
                    pltpu.async_copy(msg_hbm.at[sidx.at[j + 2]],
                                     rows_v.at[b], sems[b])
            return 0

        lax.fori_loop(0, 20, group, 0)

    plsc.subcore_barrier()

    with jax.named_scope("edge_epi"):
        pltpu.sync_copy(bidx_hbm.at[pl.ds(sid, 1)], bidx_v)
        pltpu.async_copy(agg_sh.at[bidx_v.at[0]], rows_v.at[1], s2).wait()

    @pl.when(cid == 0)
    def _():
        pltpu.sync_copy(rows_v.at[1], gagg0_hbm.at[pl.ds(sid * 128, 128)])

    @pl.when(cid == 1)
    def _():
        pltpu.sync_copy(rows_v.at[1], gagg1_hbm.at[pl.ds(sid * 128, 128)])


def _head_body(g1_ref, g2_ref, l1w_ref, l1b_ref, l2w_ref, l2b_ref,
               out_ref):
    r = jnp.maximum(g1_ref[...] + g2_ref[...], 0.0)
    m = jnp.max(r, axis=1, keepdims=True)
    e = jnp.exp(r - m)
    r = e / jnp.sum(e, axis=1, keepdims=True)
    a = r[0:B]
    b = r[B:2 * B]
    w1 = l1w_ref[...]
    dn = (((1,), (1,)), ((), ()))
    x = (lax.dot_general(a, w1[:, 0:D], dn, preferred_element_type=jnp.float32)
         + lax.dot_general(b, w1[:, D:2 * D], dn,
                           preferred_element_type=jnp.float32)
         + l1b_ref[...][None, :])
    x = jnp.where(x > 0, x, 0.01 * x)
    logits = lax.dot_general(x, l2w_ref[...], dn,
                             preferred_element_type=jnp.float32) \
        + l2b_ref[...][None, :]
    m2 = jnp.max(logits, axis=1, keepdims=True)
    e2 = jnp.exp(logits - m2)
    out_ref[...] = e2 / jnp.sum(e2, axis=1, keepdims=True)


def _head(g1, g2, L1_w, L1_b, L2_w, L2_b):
    return pl.pallas_call(
        _head_body,
        out_shape=jax.ShapeDtypeStruct((B, 2), jnp.float32),
    )(g1, g2, L1_w, L1_b, L2_w, L2_b)


@jax.jit
def kernel(batch, internal_node_ids, internal_adj, edge_index, emb, W, M, U, V,
           L1_w, L1_b, L2_w, L2_b):
    ids = internal_node_ids.astype(jnp.int32)
    adj = internal_adj.astype(jnp.int32)
    idx_comb = jnp.concatenate([ids[:, :, None], adj + K_INT], axis=2)
    idx_comb = idx_comb.reshape(N_NODES, 12)
    idx_comb = jnp.pad(idx_comb, ((0, NP - N_NODES), (0, 0)))
    idx_comb = idx_comb.reshape(NP * 12 // 96, 96)

    src = edge_index[0].astype(jnp.int32)
    dst = edge_index[1].astype(jnp.int32)
    npad = EP - N_EDGES
    pidx = jnp.arange(npad, dtype=jnp.int32)
    srcp = jnp.concatenate([src, N_NODES + pidx % (NP - N_NODES)])
    srcp = srcp.reshape(EP // 128, 128)
    dstp = jnp.concatenate([dst, N_NODES + pidx % (AGG_ROWS - N_NODES)])
    dstp = dstp.reshape(EP // 128, 128)

    bidx = jnp.concatenate([batch[:, 0], batch[:, 1]]).astype(jnp.int32)
    bidx = bidx.reshape(2 * B // 128, 128)

    tbl = _make_tables(emb, W, M)
    h_pre = _internal_kernel(tbl, idx_comb)
    msg, hu = _mid(h_pre, V, U)
    g1, g2 = _edge_kernel(msg, srcp, dstp, bidx, hu)
    return _head(g1, g2, L1_w, L1_b, L2_w, L2_b)
